# baseline (speedup 1.0000x reference)
import functools
import jax
import jax.numpy as jnp
from jax import lax
from jax.experimental import pallas as pl
from jax.experimental.pallas import tpu as pltpu
from jax.experimental.pallas import tpu_sc as plsc

NC, NS, LANES = 2, 16, 16
NW = NC * NS


def _mesh():
    return plsc.VectorSubcoreMesh(core_axis_name="c", subcore_axis_name="s",
                                  num_cores=NC, num_subcores=NS)


def _make_deg_kernel(E, NP):
    EPW = E // NW

    @functools.partial(
        pl.kernel,
        out_type=jax.ShapeDtypeStruct((NW, NP), jnp.float32),
        mesh=_mesh(),
        scratch_types=[
            pltpu.VMEM((NP,), jnp.float32),
            pltpu.VMEM((EPW,), jnp.int32),
        ],
        compiler_params=pltpu.CompilerParams(needs_layout_passes=False),
    )
    def deg_kernel(dst_hbm, out_hbm, hist_v, idx_v):
        c = lax.axis_index("c")
        s = lax.axis_index("s")
        wid = c * NS + s

        def zero_body(i, carry):
            hist_v[pl.ds(i * LANES, LANES)] = jnp.zeros((LANES,), jnp.float32)
            return carry
        lax.fori_loop(0, NP // LANES, zero_body, 0)

        pltpu.sync_copy(dst_hbm.at[pl.ds(wid * EPW, EPW)], idx_v)

        ones = jnp.full((LANES,), 1.0, jnp.float32)

        def body(i, carry):
            idx = idx_v[pl.ds(i * LANES, LANES)]
            plsc.addupdate_scatter(hist_v, [idx], ones)
            return carry
        lax.fori_loop(0, EPW // LANES, body, 0)

        pltpu.sync_copy(hist_v, out_hbm.at[wid])

    return deg_kernel


def _make_scatter_kernel(NPAD, F, NB, B):
    SLAB = NPAD // NS
    ZR = 64
    SB = NB // 2

    @functools.partial(
        pl.kernel,
        out_type=jax.ShapeDtypeStruct((NC, NPAD, F), jnp.float32),
        mesh=_mesh(),
        scratch_types=[
            pltpu.VMEM_SHARED((NPAD, F), jnp.float32),
            pltpu.VMEM((SB, B), jnp.int32),
            pltpu.VMEM((SB, B), jnp.int32),
            pltpu.VMEM((B, F), jnp.float32),
            pltpu.VMEM((B, F), jnp.float32),
            pltpu.SemaphoreType.DMA,
            pltpu.SemaphoreType.DMA,
        ],
        compiler_params=pltpu.CompilerParams(needs_layout_passes=False),
    )
    def scatter_kernel(y_hbm, src_hbm, dst_hbm, out_hbm,
                       acc_sh, src_v, dst_v, rows0, rows1, sem0, sem1):
        c = lax.axis_index("c")
        s = lax.axis_index("s")
        wid = c * NS + s

        def pair_body(i, carry):
            j = 2 * i
            pltpu.async_copy(y_hbm.at[src_v.at[j + 1]], rows1, sem1)
            pltpu.make_async_copy(y_hbm.at[src_v.at[j]], rows0, sem0).wait()
            pltpu.sync_copy(rows0, acc_sh.at[dst_v.at[j]], add=True)

            @pl.when(j + 2 < SB)
            def _():
                pltpu.async_copy(y_hbm.at[src_v.at[j + 2]], rows0, sem0)

            pltpu.make_async_copy(y_hbm.at[src_v.at[j + 1]], rows1, sem1).wait()
            pltpu.sync_copy(rows1, acc_sh.at[dst_v.at[j + 1]], add=True)
            return carry

        for h in range(2):
            pltpu.sync_copy(src_hbm.at[wid, pl.ds(h * SB, SB)], src_v)
            pltpu.sync_copy(dst_hbm.at[wid, pl.ds(h * SB, SB)], dst_v)
            pltpu.async_copy(y_hbm.at[src_v.at[0]], rows0, sem0)

            if h == 0:
                def zb(i, carry):
                    r = i // (F // LANES)
                    col = i % (F // LANES)
                    rows1[r, pl.ds(col * LANES, LANES)] = jnp.zeros(
                        (LANES,), jnp.float32)
                    return carry
                lax.fori_loop(0, ZR * (F // LANES), zb, 0)
                for k in range(SLAB // ZR):
                    pltpu.sync_copy(rows1.at[pl.ds(0, ZR)],
                                    acc_sh.at[pl.ds(s * SLAB + k * ZR, ZR)])
                plsc.subcore_barrier()

            lax.fori_loop(0, SB // 2, pair_body, 0)

        plsc.subcore_barrier()
        pltpu.sync_copy(acc_sh.at[pl.ds(s * SLAB, SLAB)],
                        out_hbm.at[c, pl.ds(s * SLAB, SLAB)])

    return scatter_kernel


def _dinv_col(degp_blk):
    d = jnp.transpose(degp_blk)
    deg = jnp.sum(d, axis=1, keepdims=True) + 1.0
    return lax.rsqrt(deg)


def _y1_body(degp_ref, x_ref, y_ref):
    dinv = _dinv_col(degp_ref[...])
    y_ref[...] = x_ref[...] * dinv


def _mid_body(degp_ref, sp_ref, y1_ref, w1_ref, b1_ref, w2_ref, y2_ref):
    dinv = _dinv_col(degp_ref[...])
    px = dinv * (sp_ref[0] + sp_ref[1] + y1_ref[...])
    h1 = jnp.dot(px, w1_ref[...], preferred_element_type=jnp.float32,
                 precision=lax.Precision.HIGHEST)
    h1 = jnp.maximum(h1 + b1_ref[...], 0.0)
    h2 = jnp.dot(h1, w2_ref[...], preferred_element_type=jnp.float32,
                 precision=lax.Precision.HIGHEST)
    y2_ref[...] = h2 * dinv


def _out_body(degp_ref, sp_ref, y2_ref, b2_ref, o_ref):
    dinv = _dinv_col(degp_ref[...])
    o_ref[...] = dinv * (sp_ref[0] + sp_ref[1] + y2_ref[...]) + b2_ref[...]


@jax.jit
def kernel(x, edge_index, W1, b1, W2, b2):
    N, F = x.shape
    E = edge_index.shape[1]
    NP = 10240
    BN = 1024
    B = 128
    EPWP = 10240
    NB = EPWP // B

    src = edge_index[0].astype(jnp.int32)
    dst = edge_index[1].astype(jnp.int32)
    pad = NW * EPWP - E
    src_p = jnp.concatenate([src, jnp.zeros((pad,), jnp.int32)])
    dst_p = jnp.concatenate(
        [dst, N + (jnp.arange(pad, dtype=jnp.int32) % (NP - N))])
    src3 = src_p.reshape(NW, NB, B)
    dst3 = dst_p.reshape(NW, NB, B)

    degp = _make_deg_kernel(E, NP)(dst)

    grid = (NP // BN,)
    degp_spec = pl.BlockSpec((NW, BN), lambda i: (0, i))
    row_spec = pl.BlockSpec((BN, F), lambda i: (i, 0))
    sp_spec = pl.BlockSpec((NC, BN, F), lambda i: (0, i, 0))

    y1 = pl.pallas_call(
        _y1_body,
        grid=grid,
        in_specs=[degp_spec, row_spec],
        out_specs=row_spec,
        out_shape=jax.ShapeDtypeStruct((N, F), jnp.float32),
    )(degp, x)

    scat = _make_scatter_kernel(NP, F, NB, B)
    s1p = scat(y1, src3, dst3)

    y2 = pl.pallas_call(
        _mid_body,
        grid=grid,
        in_specs=[degp_spec, sp_spec, row_spec,
                  pl.BlockSpec((F, 2 * F), lambda i: (0, 0)),
                  pl.BlockSpec((1, 2 * F), lambda i: (0, 0)),
                  pl.BlockSpec((2 * F, F), lambda i: (0, 0))],
        out_specs=row_spec,
        out_shape=jax.ShapeDtypeStruct((N, F), jnp.float32),
    )(degp, s1p, y1, W1, b1.reshape(1, -1), W2)

    s2p = scat(y2, src3, dst3)

    out = pl.pallas_call(
        _out_body,
        grid=grid,
        in_specs=[degp_spec, sp_spec, row_spec,
                  pl.BlockSpec((1, F), lambda i: (0, 0))],
        out_specs=row_spec,
        out_shape=jax.ShapeDtypeStruct((N, F), jnp.float32),
    )(degp, s2p, y2, b2.reshape(1, -1))

    return out


# speedup vs baseline: 1.0629x; 1.0629x over previous
import functools
import jax
import jax.numpy as jnp
from jax import lax
from jax.experimental import pallas as pl
from jax.experimental.pallas import tpu as pltpu
from jax.experimental.pallas import tpu_sc as plsc

NC, NS, LANES = 2, 16, 16
NW = NC * NS


def _mesh():
    return plsc.VectorSubcoreMesh(core_axis_name="c", subcore_axis_name="s",
                                  num_cores=NC, num_subcores=NS)


def _make_deg_kernel(E, NP):
    EPW = E // NW

    @functools.partial(
        pl.kernel,
        out_type=jax.ShapeDtypeStruct((NW, NP), jnp.float32),
        mesh=_mesh(),
        scratch_types=[
            pltpu.VMEM((NP,), jnp.float32),
            pltpu.VMEM((EPW,), jnp.int32),
        ],
        compiler_params=pltpu.CompilerParams(needs_layout_passes=False),
    )
    def deg_kernel(dst_hbm, out_hbm, hist_v, idx_v):
        c = lax.axis_index("c")
        s = lax.axis_index("s")
        wid = c * NS + s

        def zero_body(i, carry):
            hist_v[pl.ds(i * LANES, LANES)] = jnp.zeros((LANES,), jnp.float32)
            return carry
        lax.fori_loop(0, NP // LANES, zero_body, 0)

        pltpu.sync_copy(dst_hbm.at[pl.ds(wid * EPW, EPW)], idx_v)

        ones = jnp.full((LANES,), 1.0, jnp.float32)

        def body(i, carry):
            idx = idx_v[pl.ds(i * LANES, LANES)]
            plsc.addupdate_scatter(hist_v, [idx], ones)
            return carry
        lax.fori_loop(0, EPW // LANES, body, 0)

        pltpu.sync_copy(hist_v, out_hbm.at[wid])

    return deg_kernel


def _make_scatter_kernel(NPAD, F, NB, B):
    SLAB = NPAD // NS
    ZR = 64
    SB = NB // 2

    @functools.partial(
        pl.kernel,
        out_type=jax.ShapeDtypeStruct((NC, NPAD, F), jnp.float32),
        mesh=_mesh(),
        scratch_types=[
            pltpu.VMEM_SHARED((NPAD, F), jnp.float32),
            pltpu.VMEM((SB, B), jnp.int32),
            pltpu.VMEM((SB, B), jnp.int32),
            pltpu.VMEM((B, F), jnp.float32),
            pltpu.VMEM((B, F), jnp.float32),
            pltpu.SemaphoreType.DMA,
            pltpu.SemaphoreType.DMA,
        ],
        compiler_params=pltpu.CompilerParams(needs_layout_passes=False),
    )
    def scatter_kernel(y_hbm, src_hbm, dst_hbm, out_hbm,
                       acc_sh, src_v, dst_v, rows0, rows1, sem0, sem1):
        c = lax.axis_index("c")
        s = lax.axis_index("s")
        wid = c * NS + s

        def pair_body(i, carry):
            j = 2 * i
            pltpu.async_copy(y_hbm.at[src_v.at[j + 1]], rows1, sem1)
            pltpu.make_async_copy(y_hbm.at[src_v.at[j]], rows0, sem0).wait()
            pltpu.sync_copy(rows0, acc_sh.at[dst_v.at[j]], add=True)

            @pl.when(j + 2 < SB)
            def _():
                pltpu.async_copy(y_hbm.at[src_v.at[j + 2]], rows0, sem0)

            pltpu.make_async_copy(y_hbm.at[src_v.at[j + 1]], rows1, sem1).wait()
            pltpu.sync_copy(rows1, acc_sh.at[dst_v.at[j + 1]], add=True)
            return carry

        for h in range(2):
            pltpu.sync_copy(src_hbm.at[wid, pl.ds(h * SB, SB)], src_v)
            pltpu.sync_copy(dst_hbm.at[wid, pl.ds(h * SB, SB)], dst_v)
            pltpu.async_copy(y_hbm.at[src_v.at[0]], rows0, sem0)

            if h == 0:
                def zb(i, carry):
                    r = i // (F // LANES)
                    col = i % (F // LANES)
                    rows1[r, pl.ds(col * LANES, LANES)] = jnp.zeros(
                        (LANES,), jnp.float32)
                    return carry
                lax.fori_loop(0, ZR * (F // LANES), zb, 0)
                for k in range(SLAB // ZR):
                    pltpu.sync_copy(rows1.at[pl.ds(0, ZR)],
                                    acc_sh.at[pl.ds(s * SLAB + k * ZR, ZR)])
                plsc.subcore_barrier()

            lax.fori_loop(0, SB // 2, pair_body, 0)

        plsc.subcore_barrier()
        pltpu.sync_copy(acc_sh.at[pl.ds(s * SLAB, SLAB)],
                        out_hbm.at[c, pl.ds(s * SLAB, SLAB)])

    return scatter_kernel


def _dinv_col(degp_blk):
    d = jnp.transpose(degp_blk)
    deg = jnp.sum(d, axis=1, keepdims=True) + 1.0
    return lax.rsqrt(deg)


def _y1_body(degp_ref, x_ref, y_ref):
    dinv = _dinv_col(degp_ref[...])
    y_ref[...] = x_ref[...] * dinv


def _mid_body(degp_ref, sp_ref, y1_ref, w1_ref, b1_ref, w2_ref, y2_ref):
    dinv = _dinv_col(degp_ref[...])
    px = dinv * (sp_ref[0] + sp_ref[1] + y1_ref[...])
    h1 = jnp.dot(px, w1_ref[...], preferred_element_type=jnp.float32,
                 precision=lax.Precision.HIGHEST)
    h1 = jnp.maximum(h1 + b1_ref[...], 0.0)
    h2 = jnp.dot(h1, w2_ref[...], preferred_element_type=jnp.float32,
                 precision=lax.Precision.HIGHEST)
    y2_ref[...] = h2 * dinv


def _out_body(degp_ref, sp_ref, y2_ref, b2_ref, o_ref):
    dinv = _dinv_col(degp_ref[...])
    o_ref[...] = dinv * (sp_ref[0] + sp_ref[1] + y2_ref[...]) + b2_ref[...]


@jax.jit
def kernel(x, edge_index, W1, b1, W2, b2):
    N, F = x.shape
    E = edge_index.shape[1]
    NP = 10240
    BN = 1024
    B = 128
    EPWP = 10240
    NB = EPWP // B

    src = edge_index[0].astype(jnp.int32)
    dst = edge_index[1].astype(jnp.int32)
    EPW = E // NW
    ppt = EPWP - EPW
    src_p = jnp.concatenate(
        [src.reshape(NW, EPW), jnp.zeros((NW, ppt), jnp.int32)], axis=1)
    dst_p = jnp.concatenate(
        [dst.reshape(NW, EPW),
         jnp.broadcast_to(N + (jnp.arange(ppt, dtype=jnp.int32) % (NP - N)),
                          (NW, ppt))], axis=1)
    src3 = src_p.reshape(NW, NB, B)
    dst3 = dst_p.reshape(NW, NB, B)

    degp = _make_deg_kernel(E, NP)(dst)

    grid = (NP // BN,)
    degp_spec = pl.BlockSpec((NW, BN), lambda i: (0, i))
    row_spec = pl.BlockSpec((BN, F), lambda i: (i, 0))
    sp_spec = pl.BlockSpec((NC, BN, F), lambda i: (0, i, 0))

    y1 = pl.pallas_call(
        _y1_body,
        grid=grid,
        in_specs=[degp_spec, row_spec],
        out_specs=row_spec,
        out_shape=jax.ShapeDtypeStruct((N, F), jnp.float32),
    )(degp, x)

    scat = _make_scatter_kernel(NP, F, NB, B)
    s1p = scat(y1, src3, dst3)

    y2 = pl.pallas_call(
        _mid_body,
        grid=grid,
        in_specs=[degp_spec, sp_spec, row_spec,
                  pl.BlockSpec((F, 2 * F), lambda i: (0, 0)),
                  pl.BlockSpec((1, 2 * F), lambda i: (0, 0)),
                  pl.BlockSpec((2 * F, F), lambda i: (0, 0))],
        out_specs=row_spec,
        out_shape=jax.ShapeDtypeStruct((N, F), jnp.float32),
    )(degp, s1p, y1, W1, b1.reshape(1, -1), W2)

    s2p = scat(y2, src3, dst3)

    out = pl.pallas_call(
        _out_body,
        grid=grid,
        in_specs=[degp_spec, sp_spec, row_spec,
                  pl.BlockSpec((1, F), lambda i: (0, 0))],
        out_specs=row_spec,
        out_shape=jax.ShapeDtypeStruct((N, F), jnp.float32),
    )(degp, s2p, y2, b2.reshape(1, -1))

    return out


# speedup vs baseline: 2.9024x; 2.7306x over previous
import functools
import jax
import jax.numpy as jnp
from jax import lax
from jax.experimental import pallas as pl
from jax.experimental.pallas import tpu as pltpu
from jax.experimental.pallas import tpu_sc as plsc

NC, NS, LANES = 2, 16, 16
NW = NC * NS


def _mesh():
    return plsc.VectorSubcoreMesh(core_axis_name="c", subcore_axis_name="s",
                                  num_cores=NC, num_subcores=NS)


def _make_deg_kernel(E, NP):
    EPW = E // NW

    @functools.partial(
        pl.kernel,
        out_type=jax.ShapeDtypeStruct((NW, NP), jnp.float32),
        mesh=_mesh(),
        scratch_types=[
            pltpu.VMEM((NP,), jnp.float32),
            pltpu.VMEM((EPW,), jnp.int32),
        ],
        compiler_params=pltpu.CompilerParams(needs_layout_passes=False),
    )
    def deg_kernel(dst_hbm, out_hbm, hist_v, idx_v):
        c = lax.axis_index("c")
        s = lax.axis_index("s")
        wid = c * NS + s

        def zero_body(i, carry):
            hist_v[pl.ds(i * LANES, LANES)] = jnp.zeros((LANES,), jnp.float32)
            return carry
        lax.fori_loop(0, NP // LANES, zero_body, 0)

        pltpu.sync_copy(dst_hbm.at[pl.ds(wid * EPW, EPW)], idx_v)

        ones = jnp.full((LANES,), 1.0, jnp.float32)

        def body(i, carry):
            idx = idx_v[pl.ds(i * LANES, LANES)]
            plsc.addupdate_scatter(hist_v, [idx], ones)
            return carry
        lax.fori_loop(0, EPW // LANES, body, 0)

        pltpu.sync_copy(hist_v, out_hbm.at[wid])

    return deg_kernel


def _make_scatter_kernel(NPAD, F, NB, B):
    SLAB = NPAD // NS
    ZR = 64
    SB = NB // 2

    @functools.partial(
        pl.kernel,
        out_type=jax.ShapeDtypeStruct((NC, NPAD, F), jnp.float32),
        mesh=_mesh(),
        scratch_types=[
            pltpu.VMEM_SHARED((NPAD, F), jnp.float32),
            pltpu.VMEM((SB, B), jnp.int32),
            pltpu.VMEM((SB, B), jnp.int32),
            pltpu.VMEM((B, F), jnp.float32),
            pltpu.VMEM((B, F), jnp.float32),
            pltpu.SemaphoreType.DMA,
            pltpu.SemaphoreType.DMA,
        ],
        compiler_params=pltpu.CompilerParams(needs_layout_passes=False),
    )
    def scatter_kernel(y_hbm, src_hbm, dst_hbm, out_hbm,
                       acc_sh, src_v, dst_v, rows0, rows1, sem0, sem1):
        c = lax.axis_index("c")
        s = lax.axis_index("s")
        wid = c * NS + s

        def pair_body(i, carry):
            j = 2 * i
            pltpu.async_copy(y_hbm.at[src_v.at[j + 1]], rows1, sem1)
            pltpu.make_async_copy(y_hbm.at[src_v.at[j]], rows0, sem0).wait()
            pltpu.sync_copy(rows0, acc_sh.at[dst_v.at[j]], add=True)

            @pl.when(j + 2 < SB)
            def _():
                pltpu.async_copy(y_hbm.at[src_v.at[j + 2]], rows0, sem0)

            pltpu.make_async_copy(y_hbm.at[src_v.at[j + 1]], rows1, sem1).wait()
            pltpu.sync_copy(rows1, acc_sh.at[dst_v.at[j + 1]], add=True)
            return carry

        for h in range(2):
            pltpu.sync_copy(src_hbm.at[wid, h], src_v)
            pltpu.sync_copy(dst_hbm.at[wid, h], dst_v)
            pltpu.async_copy(y_hbm.at[src_v.at[0]], rows0, sem0)

            if h == 0:
                def zb(i, carry):
                    r = i // (F // LANES)
                    col = i % (F // LANES)
                    rows1[r, pl.ds(col * LANES, LANES)] = jnp.zeros(
                        (LANES,), jnp.float32)
                    return carry
                lax.fori_loop(0, ZR * (F // LANES), zb, 0)
                for k in range(SLAB // ZR):
                    pltpu.sync_copy(rows1.at[pl.ds(0, ZR)],
                                    acc_sh.at[pl.ds(s * SLAB + k * ZR, ZR)])
                plsc.subcore_barrier()

            lax.fori_loop(0, SB // 2, pair_body, 0)

        plsc.subcore_barrier()
        pltpu.sync_copy(acc_sh.at[pl.ds(s * SLAB, SLAB)],
                        out_hbm.at[c, pl.ds(s * SLAB, SLAB)])

    return scatter_kernel


def _dinv_col(degp_blk):
    d = jnp.transpose(degp_blk)
    deg = jnp.sum(d, axis=1, keepdims=True) + 1.0
    return lax.rsqrt(deg)


def _y1_body(degp_ref, x_ref, y_ref):
    dinv = _dinv_col(degp_ref[...])
    y_ref[...] = x_ref[...] * dinv


def _mid_body(degp_ref, sp_ref, y1_ref, w1_ref, b1_ref, w2_ref, y2_ref):
    dinv = _dinv_col(degp_ref[...])
    px = dinv * (sp_ref[0] + sp_ref[1] + y1_ref[...])
    h1 = jnp.dot(px, w1_ref[...], preferred_element_type=jnp.float32,
                 precision=lax.Precision.HIGHEST)
    h1 = jnp.maximum(h1 + b1_ref[...], 0.0)
    h2 = jnp.dot(h1, w2_ref[...], preferred_element_type=jnp.float32,
                 precision=lax.Precision.HIGHEST)
    y2_ref[...] = h2 * dinv


def _out_body(degp_ref, sp_ref, y2_ref, b2_ref, o_ref):
    dinv = _dinv_col(degp_ref[...])
    o_ref[...] = dinv * (sp_ref[0] + sp_ref[1] + y2_ref[...]) + b2_ref[...]


@jax.jit
def kernel(x, edge_index, W1, b1, W2, b2):
    N, F = x.shape
    E = edge_index.shape[1]
    NP = 10240
    BN = 1024
    B = 100
    EPW = E // NW
    NB = EPW // B

    src = edge_index[0].astype(jnp.int32)
    dst = edge_index[1].astype(jnp.int32)
    src3 = src.reshape(NW, 2, NB // 2, B)
    dst3 = dst.reshape(NW, 2, NB // 2, B)

    degp = _make_deg_kernel(E, NP)(dst)

    grid = (NP // BN,)
    degp_spec = pl.BlockSpec((NW, BN), lambda i: (0, i))
    row_spec = pl.BlockSpec((BN, F), lambda i: (i, 0))
    sp_spec = pl.BlockSpec((NC, BN, F), lambda i: (0, i, 0))

    y1 = pl.pallas_call(
        _y1_body,
        grid=grid,
        in_specs=[degp_spec, row_spec],
        out_specs=row_spec,
        out_shape=jax.ShapeDtypeStruct((N, F), jnp.float32),
    )(degp, x)

    scat = _make_scatter_kernel(NP, F, NB, B)
    s1p = scat(y1, src3, dst3)

    y2 = pl.pallas_call(
        _mid_body,
        grid=grid,
        in_specs=[degp_spec, sp_spec, row_spec,
                  pl.BlockSpec((F, 2 * F), lambda i: (0, 0)),
                  pl.BlockSpec((1, 2 * F), lambda i: (0, 0)),
                  pl.BlockSpec((2 * F, F), lambda i: (0, 0))],
        out_specs=row_spec,
        out_shape=jax.ShapeDtypeStruct((N, F), jnp.float32),
    )(degp, s1p, y1, W1, b1.reshape(1, -1), W2)

    s2p = scat(y2, src3, dst3)

    out = pl.pallas_call(
        _out_body,
        grid=grid,
        in_specs=[degp_spec, sp_spec, row_spec,
                  pl.BlockSpec((1, F), lambda i: (0, 0))],
        out_specs=row_spec,
        out_shape=jax.ShapeDtypeStruct((N, F), jnp.float32),
    )(degp, s2p, y2, b2.reshape(1, -1))

    return out


# speedup vs baseline: 2.9837x; 1.0280x over previous
import functools
import jax
import jax.numpy as jnp
from jax import lax
from jax.experimental import pallas as pl
from jax.experimental.pallas import tpu as pltpu
from jax.experimental.pallas import tpu_sc as plsc

NC, NS, LANES = 2, 16, 16
NW = NC * NS


def _mesh():
    return plsc.VectorSubcoreMesh(core_axis_name="c", subcore_axis_name="s",
                                  num_cores=NC, num_subcores=NS)


def _make_deg_kernel(E, NP):
    EPW = E // NW

    @functools.partial(
        pl.kernel,
        out_type=jax.ShapeDtypeStruct((NW, NP), jnp.float32),
        mesh=_mesh(),
        scratch_types=[
            pltpu.VMEM((NP,), jnp.float32),
            pltpu.VMEM((EPW,), jnp.int32),
        ],
        compiler_params=pltpu.CompilerParams(needs_layout_passes=False),
    )
    def deg_kernel(dst_hbm, out_hbm, hist_v, idx_v):
        c = lax.axis_index("c")
        s = lax.axis_index("s")
        wid = c * NS + s

        def zero_body(i, carry):
            hist_v[pl.ds(i * LANES, LANES)] = jnp.zeros((LANES,), jnp.float32)
            return carry
        lax.fori_loop(0, NP // LANES, zero_body, 0)

        pltpu.sync_copy(dst_hbm.at[pl.ds(wid * EPW, EPW)], idx_v)

        ones = jnp.full((LANES,), 1.0, jnp.float32)

        def body(i, carry):
            idx = idx_v[pl.ds(i * LANES, LANES)]
            plsc.addupdate_scatter(hist_v, [idx], ones)
            return carry
        lax.fori_loop(0, EPW // LANES, body, 0)

        pltpu.sync_copy(hist_v, out_hbm.at[wid])

    return deg_kernel


def _make_scatter_kernel(NPAD, F, NB, B):
    SLAB = NPAD // NS
    ZR = 64
    SB = NB // 2

    @functools.partial(
        pl.kernel,
        out_type=jax.ShapeDtypeStruct((NC, NPAD, F), jnp.float32),
        mesh=_mesh(),
        scratch_types=[
            pltpu.VMEM_SHARED((NPAD, F), jnp.float32),
            pltpu.VMEM((SB, B), jnp.int32),
            pltpu.VMEM((SB, B), jnp.int32),
            pltpu.VMEM((B, F), jnp.float32),
            pltpu.VMEM((B, F), jnp.float32),
            pltpu.SemaphoreType.DMA,
            pltpu.SemaphoreType.DMA,
        ],
        compiler_params=pltpu.CompilerParams(needs_layout_passes=False),
    )
    def scatter_kernel(y_hbm, src_hbm, dst_hbm, out_hbm,
                       acc_sh, src_v, dst_v, rows0, rows1, sem0, sem1):
        c = lax.axis_index("c")
        s = lax.axis_index("s")
        wid = c * NS + s

        def pair_body(i, carry):
            j = 2 * i
            pltpu.async_copy(y_hbm.at[src_v.at[j + 1]], rows1, sem1)
            pltpu.make_async_copy(y_hbm.at[src_v.at[j]], rows0, sem0).wait()
            pltpu.sync_copy(rows0, acc_sh.at[dst_v.at[j]], add=True)

            @pl.when(j + 2 < SB)
            def _():
                pltpu.async_copy(y_hbm.at[src_v.at[j + 2]], rows0, sem0)

            pltpu.make_async_copy(y_hbm.at[src_v.at[j + 1]], rows1, sem1).wait()
            pltpu.sync_copy(rows1, acc_sh.at[dst_v.at[j + 1]], add=True)
            return carry

        for h in range(2):
            pltpu.sync_copy(src_hbm.at[wid, h], src_v)
            pltpu.sync_copy(dst_hbm.at[wid, h], dst_v)
            pltpu.async_copy(y_hbm.at[src_v.at[0]], rows0, sem0)

            if h == 0:
                def zb(i, carry):
                    r = i // (F // LANES)
                    col = i % (F // LANES)
                    rows1[r, pl.ds(col * LANES, LANES)] = jnp.zeros(
                        (LANES,), jnp.float32)
                    return carry
                lax.fori_loop(0, ZR * (F // LANES), zb, 0)
                for k in range(SLAB // ZR):
                    pltpu.sync_copy(rows1.at[pl.ds(0, ZR)],
                                    acc_sh.at[pl.ds(s * SLAB + k * ZR, ZR)])
                plsc.subcore_barrier()

            lax.fori_loop(0, SB // 2, pair_body, 0)

        plsc.subcore_barrier()
        pltpu.sync_copy(acc_sh.at[pl.ds(s * SLAB, SLAB)],
                        out_hbm.at[c, pl.ds(s * SLAB, SLAB)])

    return scatter_kernel


def _dinv_col(degp_blk):
    d = jnp.transpose(degp_blk)
    deg = jnp.sum(d, axis=1, keepdims=True) + 1.0
    return lax.rsqrt(deg)


def _y1_body(degp_ref, x_ref, y_ref):
    dinv = _dinv_col(degp_ref[...])
    y_ref[...] = x_ref[...] * dinv


def _mid_body(degp_ref, sp_ref, y1_ref, w1_ref, b1_ref, w2_ref, y2_ref):
    dinv = _dinv_col(degp_ref[...])
    px = dinv * (sp_ref[0] + sp_ref[1] + y1_ref[...])
    h1 = jnp.dot(px, w1_ref[...], preferred_element_type=jnp.float32,
                 precision=lax.Precision.HIGHEST)
    h1 = jnp.maximum(h1 + b1_ref[...], 0.0)
    h2 = jnp.dot(h1, w2_ref[...], preferred_element_type=jnp.float32,
                 precision=lax.Precision.HIGHEST)
    y2_ref[...] = h2 * dinv


def _out_body(degp_ref, sp_ref, y2_ref, b2_ref, o_ref):
    dinv = _dinv_col(degp_ref[...])
    o_ref[...] = dinv * (sp_ref[0] + sp_ref[1] + y2_ref[...]) + b2_ref[...]


@jax.jit
def kernel(x, edge_index, W1, b1, W2, b2):
    N, F = x.shape
    E = edge_index.shape[1]
    NP = 10240
    BN = 1024
    B = 125
    EPW = E // NW
    NB = EPW // B

    src = edge_index[0].astype(jnp.int32)
    dst = edge_index[1].astype(jnp.int32)
    src3 = src.reshape(NW, 2, NB // 2, B)
    dst3 = dst.reshape(NW, 2, NB // 2, B)

    degp = _make_deg_kernel(E, NP)(dst)

    grid = (NP // BN,)
    degp_spec = pl.BlockSpec((NW, BN), lambda i: (0, i))
    row_spec = pl.BlockSpec((BN, F), lambda i: (i, 0))
    sp_spec = pl.BlockSpec((NC, BN, F), lambda i: (0, i, 0))

    y1 = pl.pallas_call(
        _y1_body,
        grid=grid,
        in_specs=[degp_spec, row_spec],
        out_specs=row_spec,
        out_shape=jax.ShapeDtypeStruct((N, F), jnp.float32),
    )(degp, x)

    scat = _make_scatter_kernel(NP, F, NB, B)
    s1p = scat(y1, src3, dst3)

    y2 = pl.pallas_call(
        _mid_body,
        grid=grid,
        in_specs=[degp_spec, sp_spec, row_spec,
                  pl.BlockSpec((F, 2 * F), lambda i: (0, 0)),
                  pl.BlockSpec((1, 2 * F), lambda i: (0, 0)),
                  pl.BlockSpec((2 * F, F), lambda i: (0, 0))],
        out_specs=row_spec,
        out_shape=jax.ShapeDtypeStruct((N, F), jnp.float32),
    )(degp, s1p, y1, W1, b1.reshape(1, -1), W2)

    s2p = scat(y2, src3, dst3)

    out = pl.pallas_call(
        _out_body,
        grid=grid,
        in_specs=[degp_spec, sp_spec, row_spec,
                  pl.BlockSpec((1, F), lambda i: (0, 0))],
        out_specs=row_spec,
        out_shape=jax.ShapeDtypeStruct((N, F), jnp.float32),
    )(degp, s2p, y2, b2.reshape(1, -1))

    return out


# speedup vs baseline: 3.1740x; 1.0638x over previous
import functools
import jax
import jax.numpy as jnp
from jax import lax
from jax.experimental import pallas as pl
from jax.experimental.pallas import tpu as pltpu
from jax.experimental.pallas import tpu_sc as plsc

NC, NS, LANES = 2, 16, 16
NW = NC * NS


def _mesh():
    return plsc.VectorSubcoreMesh(core_axis_name="c", subcore_axis_name="s",
                                  num_cores=NC, num_subcores=NS)


def _make_deg_kernel(E, NP):
    EPW = E // NW

    @functools.partial(
        pl.kernel,
        out_type=jax.ShapeDtypeStruct((NW, NP), jnp.float32),
        mesh=_mesh(),
        scratch_types=[
            pltpu.VMEM((NP,), jnp.float32),
            pltpu.VMEM((EPW,), jnp.int32),
        ],
        compiler_params=pltpu.CompilerParams(needs_layout_passes=False),
    )
    def deg_kernel(dst_hbm, out_hbm, hist_v, idx_v):
        c = lax.axis_index("c")
        s = lax.axis_index("s")
        wid = c * NS + s

        def zero_body(i, carry):
            hist_v[pl.ds(i * LANES, LANES)] = jnp.zeros((LANES,), jnp.float32)
            return carry
        lax.fori_loop(0, NP // LANES, zero_body, 0)

        pltpu.sync_copy(dst_hbm.at[pl.ds(wid * EPW, EPW)], idx_v)

        ones = jnp.full((LANES,), 1.0, jnp.float32)

        def body(i, carry):
            idx = idx_v[pl.ds(i * LANES, LANES)]
            plsc.addupdate_scatter(hist_v, [idx], ones)
            return carry
        lax.fori_loop(0, EPW // LANES, body, 0)

        pltpu.sync_copy(hist_v, out_hbm.at[wid])

    return deg_kernel


def _make_scatter_kernel(NPAD, F, NB, B):
    SLAB = NPAD // NS
    ZR = 64
    EPW = NB * B

    @functools.partial(
        pl.kernel,
        out_type=jax.ShapeDtypeStruct((NC, NPAD, F), jnp.float32),
        mesh=_mesh(),
        scratch_types=[
            pltpu.VMEM_SHARED((NPAD, F), jnp.float32),
            pltpu.VMEM((EPW,), jnp.int32),
            pltpu.VMEM((B, F), jnp.float32),
            pltpu.VMEM((B, F), jnp.float32),
            pltpu.VMEM((B, F), jnp.float32),
            pltpu.VMEM((1, B), jnp.int32),
            pltpu.VMEM((1, B), jnp.int32),
            pltpu.VMEM((1, B), jnp.int32),
            pltpu.VMEM((1, B), jnp.int32),
            pltpu.VMEM((1, B), jnp.int32),
            pltpu.VMEM((1, B), jnp.int32),
            pltpu.SemaphoreType.DMA,
            pltpu.SemaphoreType.DMA,
            pltpu.SemaphoreType.DMA,
            pltpu.SemaphoreType.DMA,
            pltpu.SemaphoreType.DMA,
            pltpu.SemaphoreType.DMA,
        ],
        compiler_params=pltpu.CompilerParams(needs_layout_passes=False),
    )
    def scatter_kernel(y_hbm, packed_hbm, out_hbm, acc_sh, packed_v,
                       rb0, rb1, rb2, sb0, sb1, sb2, db0, db1, db2,
                       g0, g1, g2, s0, s1, s2):
        c = lax.axis_index("c")
        s = lax.axis_index("s")
        wid = c * NS + s
        rows = (rb0, rb1, rb2)
        srcb = (sb0, sb1, sb2)
        dstb = (db0, db1, db2)
        gsem = (g0, g1, g2)
        ssem = (s0, s1, s2)
        mask = jnp.full((LANES,), 0x3FFF, jnp.int32)
        sh14 = jnp.full((LANES,), 14, jnp.int32)

        def unpack(g, m):
            for k in range(B // LANES):
                p = packed_v[pl.ds(g * B + k * LANES, LANES)]
                srcb[m][0, pl.ds(k * LANES, LANES)] = p & mask
                dstb[m][0, pl.ds(k * LANES, LANES)] = (
                    lax.shift_right_logical(p, sh14))

        def gather(g, m):
            pltpu.async_copy(y_hbm.at[srcb[m].at[0]], rows[m], gsem[m])

        def gwait(m):
            pltpu.make_async_copy(y_hbm.at[srcb[m].at[0]], rows[m],
                                  gsem[m]).wait()

        def scatter(m):
            pltpu.async_copy(rows[m], acc_sh.at[dstb[m].at[0]], ssem[m],
                             add=True)

        def swait(m):
            pltpu.make_async_copy(rows[m], acc_sh.at[dstb[m].at[0]],
                                  ssem[m]).wait()

        pltpu.sync_copy(packed_hbm.at[wid], packed_v)
        unpack(0, 0)
        gather(0, 0)
        unpack(1, 1)
        gather(1, 1)

        def zb(i, carry):
            r = i // (F // LANES)
            col = i % (F // LANES)
            rb2[r, pl.ds(col * LANES, LANES)] = jnp.zeros((LANES,), jnp.float32)
            return carry
        lax.fori_loop(0, ZR * (F // LANES), zb, 0)
        for k in range(SLAB // ZR):
            pltpu.sync_copy(rb2.at[pl.ds(0, ZR)],
                            acc_sh.at[pl.ds(s * SLAB + k * ZR, ZR)])
        plsc.subcore_barrier()

        gwait(0)
        scatter(0)
        unpack(2, 2)
        gather(2, 2)

        def body(i, carry):
            for t in range(3):
                g = 3 * i + 1 + t
                m = (1 + t) % 3
                gwait(m)
                scatter(m)
                swait((m + 2) % 3)

                @pl.when(g + 2 < NB)
                def _():
                    unpack(g + 2, (m + 2) % 3)
                    gather(g + 2, (m + 2) % 3)
            return carry
        lax.fori_loop(0, (NB - 2) // 3, body, 0)

        mlast = (NB - 1) % 3
        gwait(mlast)
        scatter(mlast)
        swait((mlast + 2) % 3)
        swait(mlast)

        plsc.subcore_barrier()
        pltpu.sync_copy(acc_sh.at[pl.ds(s * SLAB, SLAB)],
                        out_hbm.at[c, pl.ds(s * SLAB, SLAB)])

    return scatter_kernel


def _dinv_col(degp_blk):
    d = jnp.transpose(degp_blk)
    deg = jnp.sum(d, axis=1, keepdims=True) + 1.0
    return lax.rsqrt(deg)


def _y1_body(degp_ref, x_ref, y_ref):
    dinv = _dinv_col(degp_ref[...])
    y_ref[...] = x_ref[...] * dinv


def _mid_body(degp_ref, sp_ref, y1_ref, w1_ref, b1_ref, w2_ref, y2_ref):
    dinv = _dinv_col(degp_ref[...])
    px = dinv * (sp_ref[0] + sp_ref[1] + y1_ref[...])
    h1 = jnp.dot(px, w1_ref[...], preferred_element_type=jnp.float32,
                 precision=lax.Precision.HIGHEST)
    h1 = jnp.maximum(h1 + b1_ref[...], 0.0)
    h2 = jnp.dot(h1, w2_ref[...], preferred_element_type=jnp.float32,
                 precision=lax.Precision.HIGHEST)
    y2_ref[...] = h2 * dinv


def _out_body(degp_ref, sp_ref, y2_ref, b2_ref, o_ref):
    dinv = _dinv_col(degp_ref[...])
    o_ref[...] = dinv * (sp_ref[0] + sp_ref[1] + y2_ref[...]) + b2_ref[...]


@jax.jit
def kernel(x, edge_index, W1, b1, W2, b2):
    N, F = x.shape
    E = edge_index.shape[1]
    NP = 10240
    BN = 1024
    B = 80
    EPW = E // NW
    NB = EPW // B

    src = edge_index[0].astype(jnp.int32)
    dst = edge_index[1].astype(jnp.int32)
    packed = (src | (dst << 14)).reshape(NW, EPW)

    degp = _make_deg_kernel(E, NP)(dst)

    grid = (NP // BN,)
    degp_spec = pl.BlockSpec((NW, BN), lambda i: (0, i))
    row_spec = pl.BlockSpec((BN, F), lambda i: (i, 0))
    sp_spec = pl.BlockSpec((NC, BN, F), lambda i: (0, i, 0))

    y1 = pl.pallas_call(
        _y1_body,
        grid=grid,
        in_specs=[degp_spec, row_spec],
        out_specs=row_spec,
        out_shape=jax.ShapeDtypeStruct((N, F), jnp.float32),
    )(degp, x)

    scat = _make_scatter_kernel(NP, F, NB, B)
    s1p = scat(y1, packed)

    y2 = pl.pallas_call(
        _mid_body,
        grid=grid,
        in_specs=[degp_spec, sp_spec, row_spec,
                  pl.BlockSpec((F, 2 * F), lambda i: (0, 0)),
                  pl.BlockSpec((1, 2 * F), lambda i: (0, 0)),
                  pl.BlockSpec((2 * F, F), lambda i: (0, 0))],
        out_specs=row_spec,
        out_shape=jax.ShapeDtypeStruct((N, F), jnp.float32),
    )(degp, s1p, y1, W1, b1.reshape(1, -1), W2)

    s2p = scat(y2, packed)

    out = pl.pallas_call(
        _out_body,
        grid=grid,
        in_specs=[degp_spec, sp_spec, row_spec,
                  pl.BlockSpec((1, F), lambda i: (0, 0))],
        out_specs=row_spec,
        out_shape=jax.ShapeDtypeStruct((N, F), jnp.float32),
    )(degp, s2p, y2, b2.reshape(1, -1))

    return out


# speedup vs baseline: 3.4587x; 1.0897x over previous
import functools
import jax
import jax.numpy as jnp
from jax import lax
from jax.experimental import pallas as pl
from jax.experimental.pallas import tpu as pltpu
from jax.experimental.pallas import tpu_sc as plsc

NC, NS, LANES = 2, 16, 16
NW = NC * NS


def _mesh():
    return plsc.VectorSubcoreMesh(core_axis_name="c", subcore_axis_name="s",
                                  num_cores=NC, num_subcores=NS)


def _make_deg_kernel(E, NP):
    EPW = E // NW

    @functools.partial(
        pl.kernel,
        out_type=jax.ShapeDtypeStruct((NW, NP), jnp.float32),
        mesh=_mesh(),
        scratch_types=[
            pltpu.VMEM((NP,), jnp.float32),
            pltpu.VMEM((EPW,), jnp.int32),
        ],
        compiler_params=pltpu.CompilerParams(needs_layout_passes=False),
    )
    def deg_kernel(dst_hbm, out_hbm, hist_v, idx_v):
        c = lax.axis_index("c")
        s = lax.axis_index("s")
        wid = c * NS + s

        def zero_body(i, carry):
            hist_v[pl.ds(i * LANES, LANES)] = jnp.zeros((LANES,), jnp.float32)
            return carry
        lax.fori_loop(0, NP // LANES, zero_body, 0)

        pltpu.sync_copy(dst_hbm.at[pl.ds(wid * EPW, EPW)], idx_v)

        ones = jnp.full((LANES,), 1.0, jnp.float32)

        def body(i, carry):
            idx = idx_v[pl.ds(i * LANES, LANES)]
            plsc.addupdate_scatter(hist_v, [idx], ones)
            return carry
        lax.fori_loop(0, EPW // LANES, body, 0)

        pltpu.sync_copy(hist_v, out_hbm.at[wid])

    return deg_kernel


def _make_scatter_kernel(NPAD, F, NB, B):
    SLAB = NPAD // NS
    ZR = 64
    EPW = NB * B

    @functools.partial(
        pl.kernel,
        out_type=jax.ShapeDtypeStruct((NC, NPAD, F), jnp.float32),
        mesh=_mesh(),
        scratch_types=[
            pltpu.VMEM_SHARED((NPAD, F), jnp.float32),
            pltpu.VMEM((EPW,), jnp.int32),
            pltpu.VMEM((B, F), jnp.float32),
            pltpu.VMEM((B, F), jnp.float32),
            pltpu.VMEM((B, F), jnp.float32),
            pltpu.VMEM((1, B), jnp.int32),
            pltpu.VMEM((1, B), jnp.int32),
            pltpu.VMEM((1, B), jnp.int32),
            pltpu.VMEM((1, B), jnp.int32),
            pltpu.VMEM((1, B), jnp.int32),
            pltpu.VMEM((1, B), jnp.int32),
            pltpu.SemaphoreType.DMA,
            pltpu.SemaphoreType.DMA,
            pltpu.SemaphoreType.DMA,
            pltpu.SemaphoreType.DMA,
            pltpu.SemaphoreType.DMA,
            pltpu.SemaphoreType.DMA,
        ],
        compiler_params=pltpu.CompilerParams(needs_layout_passes=False),
    )
    def scatter_kernel(y_hbm, packed_hbm, out_hbm, acc_sh, packed_v,
                       rb0, rb1, rb2, sb0, sb1, sb2, db0, db1, db2,
                       g0, g1, g2, s0, s1, s2):
        c = lax.axis_index("c")
        s = lax.axis_index("s")
        wid = c * NS + s
        rows = (rb0, rb1, rb2)
        srcb = (sb0, sb1, sb2)
        dstb = (db0, db1, db2)
        gsem = (g0, g1, g2)
        ssem = (s0, s1, s2)
        mask = jnp.full((LANES,), 0x3FFF, jnp.int32)
        sh14 = jnp.full((LANES,), 14, jnp.int32)

        def unpack(g, m):
            for k in range(B // LANES):
                p = packed_v[pl.ds(g * B + k * LANES, LANES)]
                srcb[m][0, pl.ds(k * LANES, LANES)] = p & mask
                dstb[m][0, pl.ds(k * LANES, LANES)] = (
                    lax.shift_right_logical(p, sh14))

        def gather(g, m):
            pltpu.async_copy(y_hbm.at[srcb[m].at[0]], rows[m], gsem[m])

        def gwait(m):
            pltpu.make_async_copy(y_hbm.at[srcb[m].at[0]], rows[m],
                                  gsem[m]).wait()

        def scatter(m):
            pltpu.async_copy(rows[m], acc_sh.at[dstb[m].at[0]], ssem[m],
                             add=True)

        def swait(m):
            pltpu.make_async_copy(rows[m], acc_sh.at[dstb[m].at[0]],
                                  ssem[m]).wait()

        pltpu.sync_copy(packed_hbm.at[wid], packed_v)
        unpack(0, 0)
        gather(0, 0)
        unpack(1, 1)
        gather(1, 1)

        def zb(i, carry):
            r = i // (F // LANES)
            col = i % (F // LANES)
            rb2[r, pl.ds(col * LANES, LANES)] = jnp.zeros((LANES,), jnp.float32)
            return carry
        lax.fori_loop(0, ZR * (F // LANES), zb, 0)
        for k in range(SLAB // ZR):
            pltpu.sync_copy(rb2.at[pl.ds(0, ZR)],
                            acc_sh.at[pl.ds(s * SLAB + k * ZR, ZR)])
        plsc.subcore_barrier()

        gwait(0)
        scatter(0)
        unpack(2, 2)
        gather(2, 2)

        def body(i, carry):
            for t in range(3):
                g = 3 * i + 1 + t
                m = (1 + t) % 3
                gwait(m)
                scatter(m)
                swait((m + 2) % 3)

                @pl.when(g + 2 < NB)
                def _():
                    unpack(g + 2, (m + 2) % 3)
                    gather(g + 2, (m + 2) % 3)
            return carry
        lax.fori_loop(0, (NB - 2) // 3, body, 0)

        mlast = (NB - 1) % 3
        gwait(mlast)
        scatter(mlast)
        swait((mlast + 2) % 3)
        swait(mlast)

        plsc.subcore_barrier()
        pltpu.sync_copy(acc_sh.at[pl.ds(s * SLAB, SLAB)],
                        out_hbm.at[c, pl.ds(s * SLAB, SLAB)])

    return scatter_kernel


def _dinv_col(degp_blk):
    d = jnp.transpose(degp_blk)
    deg = jnp.sum(d, axis=1, keepdims=True) + 1.0
    return lax.rsqrt(deg)


def _y1_body(degp_ref, x_ref, y_ref):
    dinv = _dinv_col(degp_ref[...])
    y_ref[...] = x_ref[...] * dinv


def _mid_body(degp_ref, sp_ref, y1_ref, w1_ref, b1_ref, w2_ref, y2_ref):
    dinv = _dinv_col(degp_ref[...])
    px = dinv * (sp_ref[0] + sp_ref[1] + y1_ref[...])
    h1 = jnp.dot(px, w1_ref[...], preferred_element_type=jnp.float32)
    h1 = jnp.maximum(h1 + b1_ref[...], 0.0)
    h2 = jnp.dot(h1, w2_ref[...], preferred_element_type=jnp.float32)
    y2_ref[...] = h2 * dinv


def _out_body(degp_ref, sp_ref, y2_ref, b2_ref, o_ref):
    dinv = _dinv_col(degp_ref[...])
    o_ref[...] = dinv * (sp_ref[0] + sp_ref[1] + y2_ref[...]) + b2_ref[...]


@jax.jit
def kernel(x, edge_index, W1, b1, W2, b2):
    N, F = x.shape
    E = edge_index.shape[1]
    NP = 10240
    BN = 1024
    B = 80
    EPW = E // NW
    NB = EPW // B

    src = edge_index[0].astype(jnp.int32)
    dst = edge_index[1].astype(jnp.int32)
    packed = (src | (dst << 14)).reshape(NW, EPW)

    degp = _make_deg_kernel(E, NP)(dst)

    grid = (NP // BN,)
    degp_spec = pl.BlockSpec((NW, BN), lambda i: (0, i))
    row_spec = pl.BlockSpec((BN, F), lambda i: (i, 0))
    sp_spec = pl.BlockSpec((NC, BN, F), lambda i: (0, i, 0))

    y1 = pl.pallas_call(
        _y1_body,
        grid=grid,
        in_specs=[degp_spec, row_spec],
        out_specs=row_spec,
        out_shape=jax.ShapeDtypeStruct((N, F), jnp.float32),
    )(degp, x)

    scat = _make_scatter_kernel(NP, F, NB, B)
    s1p = scat(y1, packed)

    y2 = pl.pallas_call(
        _mid_body,
        grid=grid,
        in_specs=[degp_spec, sp_spec, row_spec,
                  pl.BlockSpec((F, 2 * F), lambda i: (0, 0)),
                  pl.BlockSpec((1, 2 * F), lambda i: (0, 0)),
                  pl.BlockSpec((2 * F, F), lambda i: (0, 0))],
        out_specs=row_spec,
        out_shape=jax.ShapeDtypeStruct((N, F), jnp.float32),
    )(degp, s1p, y1, W1, b1.reshape(1, -1), W2)

    s2p = scat(y2, packed)

    out = pl.pallas_call(
        _out_body,
        grid=grid,
        in_specs=[degp_spec, sp_spec, row_spec,
                  pl.BlockSpec((1, F), lambda i: (0, 0))],
        out_specs=row_spec,
        out_shape=jax.ShapeDtypeStruct((N, F), jnp.float32),
    )(degp, s2p, y2, b2.reshape(1, -1))

    return out


# speedup vs baseline: 3.5513x; 1.0268x over previous
import functools
import jax
import jax.numpy as jnp
from jax import lax
from jax.experimental import pallas as pl
from jax.experimental.pallas import tpu as pltpu
from jax.experimental.pallas import tpu_sc as plsc

NC, NS, LANES = 2, 16, 16
NW = NC * NS


def _mesh():
    return plsc.VectorSubcoreMesh(core_axis_name="c", subcore_axis_name="s",
                                  num_cores=NC, num_subcores=NS)


def _make_deg_kernel(E, NP):
    EPW = E // NW

    @functools.partial(
        pl.kernel,
        out_type=jax.ShapeDtypeStruct((NW, NP), jnp.float32),
        mesh=_mesh(),
        scratch_types=[
            pltpu.VMEM((NP,), jnp.float32),
            pltpu.VMEM((EPW,), jnp.int32),
        ],
        compiler_params=pltpu.CompilerParams(needs_layout_passes=False),
    )
    def deg_kernel(dst_hbm, out_hbm, hist_v, idx_v):
        c = lax.axis_index("c")
        s = lax.axis_index("s")
        wid = c * NS + s

        def zero_body(i, carry):
            hist_v[pl.ds(i * LANES, LANES)] = jnp.zeros((LANES,), jnp.float32)
            return carry
        lax.fori_loop(0, NP // LANES, zero_body, 0)

        pltpu.sync_copy(dst_hbm.at[pl.ds(wid * EPW, EPW)], idx_v)

        ones = jnp.full((LANES,), 1.0, jnp.float32)

        def body(i, carry):
            idx = idx_v[pl.ds(i * LANES, LANES)]
            plsc.addupdate_scatter(hist_v, [idx], ones)
            return carry
        lax.fori_loop(0, EPW // LANES, body, 0)

        pltpu.sync_copy(hist_v, out_hbm.at[wid])

    return deg_kernel


def _make_scatter_kernel(NPAD, F, NB, B):
    SLAB = NPAD // NS
    ZR = 64
    EPW = NB * B

    @functools.partial(
        pl.kernel,
        out_type=jax.ShapeDtypeStruct((NC, NPAD, F), jnp.float32),
        mesh=_mesh(),
        scratch_types=[
            pltpu.VMEM_SHARED((NPAD, F), jnp.float32),
            pltpu.VMEM((EPW,), jnp.int32),
            pltpu.VMEM((B, F), jnp.float32),
            pltpu.VMEM((B, F), jnp.float32),
            pltpu.VMEM((B, F), jnp.float32),
            pltpu.VMEM((1, B), jnp.int32),
            pltpu.VMEM((1, B), jnp.int32),
            pltpu.VMEM((1, B), jnp.int32),
            pltpu.VMEM((1, B), jnp.int32),
            pltpu.VMEM((1, B), jnp.int32),
            pltpu.VMEM((1, B), jnp.int32),
            pltpu.SemaphoreType.DMA,
            pltpu.SemaphoreType.DMA,
            pltpu.SemaphoreType.DMA,
            pltpu.SemaphoreType.DMA,
            pltpu.SemaphoreType.DMA,
            pltpu.SemaphoreType.DMA,
        ],
        compiler_params=pltpu.CompilerParams(needs_layout_passes=False),
    )
    def scatter_kernel(y_hbm, packed_hbm, out_hbm, acc_sh, packed_v,
                       rb0, rb1, rb2, sb0, sb1, sb2, db0, db1, db2,
                       g0, g1, g2, s0, s1, s2):
        c = lax.axis_index("c")
        s = lax.axis_index("s")
        wid = c * NS + s
        rows = (rb0, rb1, rb2)
        srcb = (sb0, sb1, sb2)
        dstb = (db0, db1, db2)
        gsem = (g0, g1, g2)
        ssem = (s0, s1, s2)
        mask = jnp.full((LANES,), 0x3FFF, jnp.int32)
        sh14 = jnp.full((LANES,), 14, jnp.int32)

        def unpack(g, m):
            for k in range(B // LANES):
                p = packed_v[pl.ds(g * B + k * LANES, LANES)]
                srcb[m][0, pl.ds(k * LANES, LANES)] = p & mask
                dstb[m][0, pl.ds(k * LANES, LANES)] = (
                    lax.shift_right_logical(p, sh14))

        def gather(g, m):
            pltpu.async_copy(y_hbm.at[srcb[m].at[0]], rows[m], gsem[m])

        def gwait(m):
            pltpu.make_async_copy(y_hbm.at[srcb[m].at[0]], rows[m],
                                  gsem[m]).wait()

        def scatter(m):
            pltpu.async_copy(rows[m], acc_sh.at[dstb[m].at[0]], ssem[m],
                             add=True)

        def swait(m):
            pltpu.make_async_copy(rows[m], acc_sh.at[dstb[m].at[0]],
                                  ssem[m]).wait()

        pltpu.sync_copy(packed_hbm.at[wid], packed_v)
        unpack(0, 0)
        gather(0, 0)
        unpack(1, 1)
        gather(1, 1)

        def zb(i, carry):
            r = i // (F // LANES)
            col = i % (F // LANES)
            rb2[r, pl.ds(col * LANES, LANES)] = jnp.zeros((LANES,), jnp.float32)
            return carry
        lax.fori_loop(0, ZR * (F // LANES), zb, 0)
        for k in range(SLAB // ZR):
            pltpu.sync_copy(rb2.at[pl.ds(0, ZR)],
                            acc_sh.at[pl.ds(s * SLAB + k * ZR, ZR)])
        plsc.subcore_barrier()

        gwait(0)
        scatter(0)
        unpack(2, 2)
        gather(2, 2)

        def body(i, carry):
            for t in range(3):
                g = 3 * i + 1 + t
                m = (1 + t) % 3
                gwait(m)
                scatter(m)
                swait((m + 2) % 3)

                @pl.when(g + 2 < NB)
                def _():
                    unpack(g + 2, (m + 2) % 3)
                    gather(g + 2, (m + 2) % 3)
            return carry
        lax.fori_loop(0, (NB - 2) // 3, body, 0)

        mlast = (NB - 1) % 3
        gwait(mlast)
        scatter(mlast)
        swait((mlast + 2) % 3)
        swait(mlast)

        plsc.subcore_barrier()
        pltpu.sync_copy(acc_sh.at[pl.ds(s * SLAB, SLAB)],
                        out_hbm.at[c, pl.ds(s * SLAB, SLAB)])

    return scatter_kernel


def _dinv_col(degp_blk):
    d = jnp.transpose(degp_blk)
    deg = jnp.sum(d, axis=1, keepdims=True) + 1.0
    return lax.rsqrt(deg)


def _y1_body(degp_ref, x_ref, y_ref):
    dinv = _dinv_col(degp_ref[...])
    y_ref[...] = x_ref[...] * dinv


def _mid_body(degp_ref, sp_ref, y1_ref, w1_ref, b1_ref, w2_ref, y2_ref):
    dinv = _dinv_col(degp_ref[...])
    px = dinv * (sp_ref[0] + sp_ref[1] + y1_ref[...])
    h1 = jnp.dot(px, w1_ref[...], preferred_element_type=jnp.float32)
    h1 = jnp.maximum(h1 + b1_ref[...], 0.0)
    h2 = jnp.dot(h1, w2_ref[...], preferred_element_type=jnp.float32)
    y2_ref[...] = h2 * dinv


def _out_body(degp_ref, sp_ref, y2_ref, b2_ref, o_ref):
    dinv = _dinv_col(degp_ref[...])
    o_ref[...] = dinv * (sp_ref[0] + sp_ref[1] + y2_ref[...]) + b2_ref[...]


@jax.jit
def kernel(x, edge_index, W1, b1, W2, b2):
    N, F = x.shape
    E = edge_index.shape[1]
    NP = 10240
    BN = 2048
    B = 80
    EPW = E // NW
    NB = EPW // B

    src = edge_index[0].astype(jnp.int32)
    dst = edge_index[1].astype(jnp.int32)
    packed = (src | (dst << 14)).reshape(NW, EPW)

    degp = _make_deg_kernel(E, NP)(dst)

    grid = (NP // BN,)
    degp_spec = pl.BlockSpec((NW, BN), lambda i: (0, i))
    row_spec = pl.BlockSpec((BN, F), lambda i: (i, 0))
    sp_spec = pl.BlockSpec((NC, BN, F), lambda i: (0, i, 0))

    y1 = pl.pallas_call(
        _y1_body,
        grid=grid,
        in_specs=[degp_spec, row_spec],
        out_specs=row_spec,
        out_shape=jax.ShapeDtypeStruct((N, F), jnp.float32),
    )(degp, x)

    scat = _make_scatter_kernel(NP, F, NB, B)
    s1p = scat(y1, packed)

    y2 = pl.pallas_call(
        _mid_body,
        grid=grid,
        in_specs=[degp_spec, sp_spec, row_spec,
                  pl.BlockSpec((F, 2 * F), lambda i: (0, 0)),
                  pl.BlockSpec((1, 2 * F), lambda i: (0, 0)),
                  pl.BlockSpec((2 * F, F), lambda i: (0, 0))],
        out_specs=row_spec,
        out_shape=jax.ShapeDtypeStruct((N, F), jnp.float32),
    )(degp, s1p, y1, W1, b1.reshape(1, -1), W2)

    s2p = scat(y2, packed)

    out = pl.pallas_call(
        _out_body,
        grid=grid,
        in_specs=[degp_spec, sp_spec, row_spec,
                  pl.BlockSpec((1, F), lambda i: (0, 0))],
        out_specs=row_spec,
        out_shape=jax.ShapeDtypeStruct((N, F), jnp.float32),
    )(degp, s2p, y2, b2.reshape(1, -1))

    return out
